# baseline (speedup 1.0000x reference)
import jax
import jax.numpy as jnp
from jax.experimental import pallas as pl

_B = 8
_P = 1024
_BR = 512
_EPS = 1e-5


def _fwd_kernel(x_ref, w_ref, b_ref, m_ref, y_ref, s_ref, ss_ref, c_ref):
    y = jnp.dot(x_ref[...], w_ref[...], preferred_element_type=jnp.float32)
    y = jnp.maximum(y + b_ref[...], 0.0)
    y_ref[...] = y
    m = m_ref[...]
    ym = y * m

    @pl.when(pl.program_id(0) == 0)
    def _init():
        s_ref[...] = jnp.zeros_like(s_ref)
        ss_ref[...] = jnp.zeros_like(ss_ref)
        c_ref[...] = jnp.zeros_like(c_ref)

    s_ref[...] += jnp.sum(ym, axis=0, keepdims=True)
    ss_ref[...] += jnp.sum(ym * y, axis=0, keepdims=True)
    c_ref[...] += jnp.sum(m)


def _bn_kernel(y_ref, s_ref, ss_ref, c_ref, g_ref, bt_ref, o_ref):
    c = jnp.maximum(c_ref[0, 0], 1.0)
    mean = s_ref[...] / c
    var = jnp.maximum(ss_ref[...] / c - mean * mean, 0.0)
    rstd = jax.lax.rsqrt(var + _EPS)
    o_ref[...] = (y_ref[...] - mean) * rstd * g_ref[...] + bt_ref[...]


def _dense_bn(x, lp, mask):
    n, cin = x.shape
    w = lp["W"]
    cout = w.shape[1]
    np_ = -(-n // _BR) * _BR
    cip = -(-cin // 128) * 128
    cop = -(-cout // 128) * 128
    xp = jnp.pad(x, ((0, np_ - n), (0, cip - cin)))
    wp = jnp.pad(w, ((0, cip - cin), (0, cop - cout)))
    bp = jnp.pad(lp["b"], (0, cop - cout)).reshape(1, cop)
    gp = jnp.pad(lp["gamma"], (0, cop - cout)).reshape(1, cop)
    btp = jnp.pad(lp["beta"], (0, cop - cout)).reshape(1, cop)
    mp = jnp.pad(mask.astype(jnp.float32), (0, np_ - n)).reshape(np_, 1)
    grid = np_ // _BR

    y, s, ss, c = pl.pallas_call(
        _fwd_kernel,
        grid=(grid,),
        in_specs=[
            pl.BlockSpec((_BR, cip), lambda i: (i, 0)),
            pl.BlockSpec((cip, cop), lambda i: (0, 0)),
            pl.BlockSpec((1, cop), lambda i: (0, 0)),
            pl.BlockSpec((_BR, 1), lambda i: (i, 0)),
        ],
        out_specs=[
            pl.BlockSpec((_BR, cop), lambda i: (i, 0)),
            pl.BlockSpec((1, cop), lambda i: (0, 0)),
            pl.BlockSpec((1, cop), lambda i: (0, 0)),
            pl.BlockSpec((1, 1), lambda i: (0, 0)),
        ],
        out_shape=[
            jax.ShapeDtypeStruct((np_, cop), jnp.float32),
            jax.ShapeDtypeStruct((1, cop), jnp.float32),
            jax.ShapeDtypeStruct((1, cop), jnp.float32),
            jax.ShapeDtypeStruct((1, 1), jnp.float32),
        ],
    )(xp, wp, bp, mp)

    out = pl.pallas_call(
        _bn_kernel,
        grid=(grid,),
        in_specs=[
            pl.BlockSpec((_BR, cop), lambda i: (i, 0)),
            pl.BlockSpec((1, cop), lambda i: (0, 0)),
            pl.BlockSpec((1, cop), lambda i: (0, 0)),
            pl.BlockSpec((1, 1), lambda i: (0, 0)),
            pl.BlockSpec((1, cop), lambda i: (0, 0)),
            pl.BlockSpec((1, cop), lambda i: (0, 0)),
        ],
        out_specs=pl.BlockSpec((_BR, cop), lambda i: (i, 0)),
        out_shape=jax.ShapeDtypeStruct((np_, cop), jnp.float32),
    )(y, s, ss, c, gp, btp)
    return out[:n, :cout]


def _mlp_p(x, layers, mask):
    shp = x.shape
    x2 = x.reshape(-1, shp[-1])
    m2 = mask.reshape(-1)
    for lp in layers:
        x2 = _dense_bn(x2, lp, m2)
    return x2.reshape(*shp[:-1], x2.shape[-1])


def _gather(arr, idx):
    return jax.vmap(lambda a, i: a[i])(arr, idx)


def _fps(pos_b, s):
    pg = jax.lax.stop_gradient(pos_b)
    d2 = jnp.sum((pg[:, :, None, :] - pg[:, None, :, :]) ** 2, axis=-1)
    sel = jnp.zeros((pg.shape[0], s), dtype=jnp.int32)
    mind = d2[:, 0, :]

    def body(i, carry):
        mind, sel = carry
        nxt = jnp.argmax(mind, axis=1).astype(jnp.int32)
        sel = sel.at[:, i].set(nxt)
        dn = jnp.take_along_axis(d2, nxt[:, None, None], axis=1)[:, 0, :]
        return jnp.minimum(mind, dn), sel

    mind, sel = jax.lax.fori_loop(1, s, body, (mind, sel))
    return sel


def _point_conv(x, pos, r, k, layers):
    d2 = jnp.sum((pos[:, :, None, :] - pos[:, None, :, :]) ** 2, axis=-1)
    dm = jnp.where(d2 <= r * r, jax.lax.stop_gradient(d2), jnp.inf)
    neg, idx = jax.lax.top_k(-dm, k)
    valid = jnp.isfinite(neg)
    pos_j = _gather(pos, idx)
    rel = pos_j - pos[:, :, None, :]
    x_j = _gather(x, idx)
    h = jnp.concatenate([x_j, rel], axis=-1)
    h = _mlp_p(h, layers, valid)
    h = jnp.where(valid[..., None], h, -jnp.inf)
    return jnp.max(h, axis=2)


def _knn_interp1(feat_lr, pos_lr, pos_hr):
    d2 = jnp.sum((pos_hr[:, :, None, :] - pos_lr[:, None, :, :]) ** 2, axis=-1)
    nn_idx = jnp.argmin(d2, axis=2)
    return _gather(feat_lr, nn_idx)


def _branch(f, p, s, params):
    sel = _fps(p, s)
    pos_lr = _gather(p, sel)
    x_lr = _gather(f, sel)
    x1 = _point_conv(x_lr, pos_lr, 0.4, 16, params["conv1"])
    x2 = _point_conv(x1, pos_lr, 0.9, 32, params["conv2"])
    lr_feat = jnp.concatenate([x2, pos_lr], axis=-1)
    interp = _knn_interp1(lr_feat, pos_lr, p)
    return jnp.concatenate([f, p, interp], axis=-1)


def kernel(features, pos, params, batch_idx):
    f = features.reshape(_B, _P, -1)
    p = pos.reshape(_B, _P, 3)
    hr = _branch(f, p, 512, params)
    mr = _branch(f, p, 256, params)
    lr = _branch(f, p, 128, params)
    x = jnp.concatenate([hr, mr, lr, p], axis=-1).reshape(_B * _P, -1)
    mask = jnp.ones((x.shape[0],), dtype=jnp.float32)
    return _mlp_p(x, params["head"], mask)


# speedup vs baseline: 1.0595x; 1.0595x over previous
import jax
import jax.numpy as jnp
from jax.experimental import pallas as pl

_B = 8
_P = 1024
_BR = 512
_EPS = 1e-5


def _fwd_kernel(x_ref, w_ref, b_ref, m_ref, y_ref, s_ref, ss_ref, c_ref):
    y = jnp.dot(x_ref[...], w_ref[...], preferred_element_type=jnp.float32)
    y = jnp.maximum(y + b_ref[...], 0.0)
    y_ref[...] = y
    m = m_ref[...]
    ym = y * m

    @pl.when(pl.program_id(0) == 0)
    def _init():
        s_ref[...] = jnp.zeros_like(s_ref)
        ss_ref[...] = jnp.zeros_like(ss_ref)
        c_ref[...] = jnp.zeros_like(c_ref)

    s_ref[...] += jnp.sum(ym, axis=0, keepdims=True)
    ss_ref[...] += jnp.sum(ym * y, axis=0, keepdims=True)
    c_ref[...] += jnp.sum(m)


def _bn_math(y, s, ss, c, g, bt):
    cnt = jnp.maximum(c, 1.0)
    mean = s / cnt
    var = jnp.maximum(ss / cnt - mean * mean, 0.0)
    rstd = jax.lax.rsqrt(var + _EPS)
    return (y - mean) * rstd * g + bt


def _bn_kernel(y_ref, s_ref, ss_ref, c_ref, g_ref, bt_ref, o_ref):
    o_ref[...] = _bn_math(
        y_ref[...], s_ref[...], ss_ref[...], c_ref[0, 0], g_ref[...], bt_ref[...]
    )


def _fused_kernel(x_ref, ps_ref, pss_ref, pc_ref, pg_ref, pbt_ref,
                  w_ref, b_ref, m_ref, y_ref, s_ref, ss_ref, c_ref):
    x = _bn_math(x_ref[...], ps_ref[...], pss_ref[...], pc_ref[0, 0],
                 pg_ref[...], pbt_ref[...])
    y = jnp.dot(x, w_ref[...], preferred_element_type=jnp.float32)
    y = jnp.maximum(y + b_ref[...], 0.0)
    y_ref[...] = y
    m = m_ref[...]
    ym = y * m

    @pl.when(pl.program_id(0) == 0)
    def _init():
        s_ref[...] = jnp.zeros_like(s_ref)
        ss_ref[...] = jnp.zeros_like(ss_ref)
        c_ref[...] = jnp.zeros_like(c_ref)

    s_ref[...] += jnp.sum(ym, axis=0, keepdims=True)
    ss_ref[...] += jnp.sum(ym * y, axis=0, keepdims=True)
    c_ref[...] += jnp.sum(m)


def _mlp_p(x, layers, mask):
    shp = x.shape
    x2 = x.reshape(-1, shp[-1])
    n, cin = x2.shape
    np_ = -(-n // _BR) * _BR
    grid = np_ // _BR
    mp = jnp.pad(mask.reshape(-1).astype(jnp.float32), (0, np_ - n)).reshape(np_, 1)
    cip = -(-cin // 128) * 128
    xp = jnp.pad(x2, ((0, np_ - n), (0, cip - cin)))

    stat_specs = [
        pl.BlockSpec((1, cip), lambda i: (0, 0)),
        pl.BlockSpec((1, cip), lambda i: (0, 0)),
        pl.BlockSpec((1, 1), lambda i: (0, 0)),
    ]
    prev = None
    for lp in layers:
        cout = lp["W"].shape[1]
        cop = -(-cout // 128) * 128
        wp = jnp.pad(lp["W"], ((0, cip - lp["W"].shape[0]), (0, cop - cout)))
        bp = jnp.pad(lp["b"], (0, cop - cout)).reshape(1, cop)
        gp = jnp.pad(lp["gamma"], (0, cop - cout)).reshape(1, cop)
        btp = jnp.pad(lp["beta"], (0, cop - cout)).reshape(1, cop)
        out_specs = [
            pl.BlockSpec((_BR, cop), lambda i: (i, 0)),
            pl.BlockSpec((1, cop), lambda i: (0, 0)),
            pl.BlockSpec((1, cop), lambda i: (0, 0)),
            pl.BlockSpec((1, 1), lambda i: (0, 0)),
        ]
        out_shape = [
            jax.ShapeDtypeStruct((np_, cop), jnp.float32),
            jax.ShapeDtypeStruct((1, cop), jnp.float32),
            jax.ShapeDtypeStruct((1, cop), jnp.float32),
            jax.ShapeDtypeStruct((1, 1), jnp.float32),
        ]
        mat_specs = [
            pl.BlockSpec((cip, cop), lambda i: (0, 0)),
            pl.BlockSpec((1, cop), lambda i: (0, 0)),
            pl.BlockSpec((_BR, 1), lambda i: (i, 0)),
        ]
        if prev is None:
            y, s, ss, c = pl.pallas_call(
                _fwd_kernel,
                grid=(grid,),
                in_specs=[pl.BlockSpec((_BR, cip), lambda i: (i, 0))] + mat_specs,
                out_specs=out_specs,
                out_shape=out_shape,
            )(xp, wp, bp, mp)
        else:
            py, ps, pss, pc, pgp, pbtp = prev
            y, s, ss, c = pl.pallas_call(
                _fused_kernel,
                grid=(grid,),
                in_specs=[pl.BlockSpec((_BR, cip), lambda i: (i, 0))]
                + stat_specs
                + [pl.BlockSpec((1, cip), lambda i: (0, 0))] * 2
                + mat_specs,
                out_specs=out_specs,
                out_shape=out_shape,
            )(py, ps, pss, pc, pgp, pbtp, wp, bp, mp)
        prev = (y, s, ss, c, gp, btp)
        cip = cop
        stat_specs = [
            pl.BlockSpec((1, cop), lambda i: (0, 0)),
            pl.BlockSpec((1, cop), lambda i: (0, 0)),
            pl.BlockSpec((1, 1), lambda i: (0, 0)),
        ]

    y, s, ss, c, gp, btp = prev
    out = pl.pallas_call(
        _bn_kernel,
        grid=(grid,),
        in_specs=[pl.BlockSpec((_BR, cip), lambda i: (i, 0))]
        + stat_specs
        + [pl.BlockSpec((1, cip), lambda i: (0, 0))] * 2,
        out_specs=pl.BlockSpec((_BR, cip), lambda i: (i, 0)),
        out_shape=jax.ShapeDtypeStruct((np_, cip), jnp.float32),
    )(y, s, ss, c, gp, btp)
    return out[:n, :cout].reshape(*shp[:-1], cout)


def _gather(arr, idx):
    return jax.vmap(lambda a, i: a[i])(arr, idx)


def _fps(pos_b, s):
    pg = jax.lax.stop_gradient(pos_b)
    d2 = jnp.sum((pg[:, :, None, :] - pg[:, None, :, :]) ** 2, axis=-1)
    sel = jnp.zeros((pg.shape[0], s), dtype=jnp.int32)
    mind = d2[:, 0, :]

    def body(i, carry):
        mind, sel = carry
        nxt = jnp.argmax(mind, axis=1).astype(jnp.int32)
        sel = sel.at[:, i].set(nxt)
        dn = jnp.take_along_axis(d2, nxt[:, None, None], axis=1)[:, 0, :]
        return jnp.minimum(mind, dn), sel

    mind, sel = jax.lax.fori_loop(1, s, body, (mind, sel))
    return sel


def _point_conv(x, pos, r, k, layers):
    d2 = jnp.sum((pos[:, :, None, :] - pos[:, None, :, :]) ** 2, axis=-1)
    dm = jnp.where(d2 <= r * r, jax.lax.stop_gradient(d2), jnp.inf)
    neg, idx = jax.lax.top_k(-dm, k)
    valid = jnp.isfinite(neg)
    pos_j = _gather(pos, idx)
    rel = pos_j - pos[:, :, None, :]
    x_j = _gather(x, idx)
    h = jnp.concatenate([x_j, rel], axis=-1)
    h = _mlp_p(h, layers, valid)
    h = jnp.where(valid[..., None], h, -jnp.inf)
    return jnp.max(h, axis=2)


def _knn_interp1(feat_lr, pos_lr, pos_hr):
    d2 = jnp.sum((pos_hr[:, :, None, :] - pos_lr[:, None, :, :]) ** 2, axis=-1)
    nn_idx = jnp.argmin(d2, axis=2)
    return _gather(feat_lr, nn_idx)


def _branch(f, p, s, params):
    sel = _fps(p, s)
    pos_lr = _gather(p, sel)
    x_lr = _gather(f, sel)
    x1 = _point_conv(x_lr, pos_lr, 0.4, 16, params["conv1"])
    x2 = _point_conv(x1, pos_lr, 0.9, 32, params["conv2"])
    lr_feat = jnp.concatenate([x2, pos_lr], axis=-1)
    interp = _knn_interp1(lr_feat, pos_lr, p)
    return jnp.concatenate([f, p, interp], axis=-1)


def kernel(features, pos, params, batch_idx):
    f = features.reshape(_B, _P, -1)
    p = pos.reshape(_B, _P, 3)
    hr = _branch(f, p, 512, params)
    mr = _branch(f, p, 256, params)
    lr = _branch(f, p, 128, params)
    x = jnp.concatenate([hr, mr, lr, p], axis=-1).reshape(_B * _P, -1)
    mask = jnp.ones((x.shape[0],), dtype=jnp.float32)
    return _mlp_p(x, params["head"], mask)
